# Initial kernel scaffold; baseline (speedup 1.0000x reference)
import jax
import jax.numpy as jnp
from jax.experimental import pallas as pl


def kernel(x, edge_index, edge_attr, W0, b0, root0, g0, be0, W1, b1, root1, g1, be1):
    raise NotImplementedError("write your pallas kernel here")


# speedup vs baseline: 9.4900x; 9.4900x over previous
import functools

import jax
import jax.numpy as jnp
from jax import lax
from jax.experimental import pallas as pl
from jax.experimental.pallas import tpu as pltpu
from jax.experimental.pallas import tpu_sc as plsc

N = 10000
E = 320000
D = 128
A = 8

NC = 2
NS = 16
NW = NC * NS
EPW = E // NW
CH = 80
NCHUNK = EPW // CH
NP = 10240
RPT = NP // NS
SR = 128

_BN_SCALE = (1.0 + 1e-5) ** -0.5

_sc_mesh = plsc.VectorSubcoreMesh(core_axis_name="c", subcore_axis_name="s")


@functools.partial(
    pl.kernel,
    out_type=jax.ShapeDtypeStruct((NW, NP), jnp.float32),
    mesh=_sc_mesh,
    scratch_types=[
        pltpu.VMEM((NP,), jnp.float32),
        pltpu.VMEM((EPW,), jnp.int32),
    ],
    compiler_params=pltpu.CompilerParams(needs_layout_passes=False),
)
def _sc_hist(row_hbm, out_hbm, hist, idxs):
    cid = lax.axis_index("c")
    sid = lax.axis_index("s")
    wid = sid * NC + cid
    zero16 = jnp.zeros((16,), jnp.float32)

    def zbody(k, carry):
        hist[pl.ds(k * 16, 16)] = zero16
        return carry

    lax.fori_loop(0, NP // 16, zbody, 0)
    pltpu.sync_copy(row_hbm.at[pl.ds(wid * EPW, EPW)], idxs)
    ones16 = jnp.ones((16,), jnp.float32)

    def body(k, carry):
        idxv = idxs[pl.ds(k * 16, 16)]
        plsc.addupdate_scatter(hist, [idxv], ones16)
        return carry

    lax.fori_loop(0, EPW // 16, body, 0)
    pltpu.sync_copy(hist, out_hbm.at[wid])


@functools.partial(
    pl.kernel,
    out_type=jax.ShapeDtypeStruct((2 * NP, D), jnp.float32),
    mesh=_sc_mesh,
    scratch_types=[
        pltpu.VMEM_SHARED((NP, D), jnp.float32),
        pltpu.VMEM((SR, D), jnp.float32),
        pltpu.VMEM((CH,), jnp.int32),
        pltpu.VMEM((CH,), jnp.int32),
        pltpu.VMEM((CH, D), jnp.float32),
        pltpu.SemaphoreType.DMA,
    ],
    compiler_params=pltpu.CompilerParams(use_tc_tiling_on_sc=False),
)
def _sc_agg(z_hbm, gidx_hbm, col_hbm, zeros_hbm, out_hbm,
            acc, stage, idxb, colb, rows, sem):
    cid = lax.axis_index("c")
    sid = lax.axis_index("s")
    wid = sid * NC + cid
    base = wid * EPW
    pltpu.sync_copy(zeros_hbm.at[pl.ds(0, SR)], stage)

    def zbody(t, carry):
        pltpu.sync_copy(stage, acc.at[pl.ds(sid * RPT + t * SR, SR)])
        return carry

    lax.fori_loop(0, RPT // SR, zbody, 0)
    plsc.subcore_barrier()

    def body(j, carry):
        pltpu.sync_copy(gidx_hbm.at[pl.ds(base + j * CH, CH)], idxb)
        pltpu.sync_copy(col_hbm.at[pl.ds(base + j * CH, CH)], colb)
        pltpu.async_copy(z_hbm.at[idxb], rows, sem).wait()
        pltpu.sync_copy(rows, acc.at[colb], add=True)
        return carry

    lax.fori_loop(0, NCHUNK, body, 0)
    plsc.subcore_barrier()

    def wbody(t, carry):
        pltpu.sync_copy(acc.at[pl.ds(sid * RPT + t * SR, SR)], stage)
        pltpu.sync_copy(stage, out_hbm.at[pl.ds(cid * NP + sid * RPT + t * SR, SR)])
        return carry

    lax.fori_loop(0, RPT // SR, wbody, 0)


_BR = 400


def _t1_body(h_ref, w_ref, b_ref, root_ref, dis_ref, inv_ref, z_ref, self_ref):
    x = lax.dot_general(h_ref[...], w_ref[...], (((1,), (1,)), ((), ())),
                        preferred_element_type=jnp.float32) + b_ref[...]
    self_ref[...] = jnp.maximum(x + root_ref[...], 0.0) * inv_ref[...]
    aval = lax.broadcasted_iota(jnp.int32, (_BR, A, D), 1).astype(jnp.float32)
    dis = dis_ref[...]
    z_ref[...] = jnp.maximum(x[:, None, :] + aval, 0.0) * dis[:, :, None]


def _t1(h, w, b, root, dis, inv):
    grid = (N // _BR,)
    return pl.pallas_call(
        _t1_body,
        grid=grid,
        in_specs=[
            pl.BlockSpec((_BR, D), lambda i: (i, 0)),
            pl.BlockSpec((D, D), lambda i: (0, 0)),
            pl.BlockSpec((1, D), lambda i: (0, 0)),
            pl.BlockSpec((1, D), lambda i: (0, 0)),
            pl.BlockSpec((_BR, 1), lambda i: (i, 0)),
            pl.BlockSpec((_BR, 1), lambda i: (i, 0)),
        ],
        out_specs=[
            pl.BlockSpec((_BR, A, D), lambda i: (i, 0, 0)),
            pl.BlockSpec((_BR, D), lambda i: (i, 0)),
        ],
        out_shape=[
            jax.ShapeDtypeStruct((N, A, D), jnp.float32),
            jax.ShapeDtypeStruct((N, D), jnp.float32),
        ],
    )(h, w, b, root, dis, inv)


def _t2_body(p0_ref, p1_ref, dis_ref, self_ref, g_ref, be_ref, o_ref, *, relu):
    y = (p0_ref[...] + p1_ref[...]) * dis_ref[...] + self_ref[...]
    y = y * (g_ref[...] * _BN_SCALE) + be_ref[...]
    if relu:
        y = jnp.maximum(y, 0.0)
    o_ref[...] = y


_BR2 = 80


def _t2(p, dis, selft, g, be, relu):
    grid = (N // _BR2,)
    nb = NP // _BR2
    return pl.pallas_call(
        functools.partial(_t2_body, relu=relu),
        grid=grid,
        in_specs=[
            pl.BlockSpec((_BR2, D), lambda i: (i, 0)),
            pl.BlockSpec((_BR2, D), lambda i, _nb=nb: (i + _nb, 0)),
            pl.BlockSpec((_BR2, 1), lambda i: (i, 0)),
            pl.BlockSpec((_BR2, D), lambda i: (i, 0)),
            pl.BlockSpec((1, D), lambda i: (0, 0)),
            pl.BlockSpec((1, D), lambda i: (0, 0)),
        ],
        out_specs=pl.BlockSpec((_BR2, D), lambda i: (i, 0)),
        out_shape=jax.ShapeDtypeStruct((N, D), jnp.float32),
    )(p, p, dis, selft, g, be)


def kernel(x, edge_index, edge_attr, W0, b0, root0, g0, be0,
           W1, b1, root1, g1, be1):
    row = edge_index[0]
    col = edge_index[1]
    gidx = row * A + edge_attr[:, 0]

    zerosD = jnp.zeros((NP, D), jnp.float32)

    hist = _sc_hist(row)
    deg = jnp.sum(hist, axis=0)[:N] + 1.0
    dis = (deg ** -0.5).reshape(N, 1)
    inv = (1.0 / deg).reshape(N, 1)

    b0r = b0.reshape(1, D)
    g0r = g0.reshape(1, D)
    be0r = be0.reshape(1, D)
    b1r = b1.reshape(1, D)
    g1r = g1.reshape(1, D)
    be1r = be1.reshape(1, D)

    z0, self0 = _t1(x, W0, b0r, root0, dis, inv)
    p0 = _sc_agg(z0.reshape(N * A, D), gidx, col, zerosD)
    h1 = _t2(p0, dis, self0, g0r, be0r, relu=True)
    z1, self1 = _t1(h1, W1, b1r, root1, dis, inv)
    p1 = _sc_agg(z1.reshape(N * A, D), gidx, col, zerosD)
    h2 = _t2(p1, dis, self1, g1r, be1r, relu=False)
    return h2


# speedup vs baseline: 16.7395x; 1.7639x over previous
import functools

import jax
import jax.numpy as jnp
from jax import lax
from jax.experimental import pallas as pl
from jax.experimental.pallas import tpu as pltpu
from jax.experimental.pallas import tpu_sc as plsc

N = 10000
E = 320000
D = 128
A = 8

NC = 2
NS = 16
NW = NC * NS
EPW = E // NW
CH = 80
NCHUNK = EPW // CH
NP = 10240
RPT = NP // NS
SR = 32

_BN_SCALE = (1.0 + 1e-5) ** -0.5

_sc_mesh = plsc.VectorSubcoreMesh(core_axis_name="c", subcore_axis_name="s")


@functools.partial(
    pl.kernel,
    out_type=jax.ShapeDtypeStruct((NW, NP), jnp.float32),
    mesh=_sc_mesh,
    scratch_types=[
        pltpu.VMEM((NP,), jnp.float32),
        pltpu.VMEM((EPW,), jnp.int32),
    ],
    compiler_params=pltpu.CompilerParams(needs_layout_passes=False),
)
def _sc_hist(row_hbm, out_hbm, hist, idxs):
    cid = lax.axis_index("c")
    sid = lax.axis_index("s")
    wid = sid * NC + cid
    zero16 = jnp.zeros((16,), jnp.float32)

    def zbody(k, carry):
        hist[pl.ds(k * 16, 16)] = zero16
        return carry

    lax.fori_loop(0, NP // 16, zbody, 0)
    pltpu.sync_copy(row_hbm.at[pl.ds(wid * EPW, EPW)], idxs)
    ones16 = jnp.ones((16,), jnp.float32)

    def body(k, carry):
        idxv = idxs[pl.ds(k * 16, 16)]
        plsc.addupdate_scatter(hist, [idxv], ones16)
        return carry

    lax.fori_loop(0, EPW // 16, body, 0)
    pltpu.sync_copy(hist, out_hbm.at[wid])


@functools.partial(
    pl.kernel,
    out_type=jax.ShapeDtypeStruct((2 * NP, D), jnp.float32),
    mesh=_sc_mesh,
    scratch_types=[
        pltpu.VMEM_SHARED((NP, D), jnp.float32),
        pltpu.VMEM((SR, D), jnp.float32),
        pltpu.VMEM((EPW,), jnp.int32),
        pltpu.VMEM((NCHUNK, CH), jnp.int32),
        pltpu.VMEM((CH, D), jnp.float32),
        pltpu.VMEM((CH, D), jnp.float32),
        pltpu.SemaphoreType.DMA,
        pltpu.SemaphoreType.DMA,
    ],
    compiler_params=pltpu.CompilerParams(use_tc_tiling_on_sc=False),
)
def _sc_agg(z_hbm, gidx_hbm, col_hbm, zeros_hbm, out_hbm,
            acc, stage, idxall, colall, rows0, rows1, sem0, sem1):
    cid = lax.axis_index("c")
    sid = lax.axis_index("s")
    wid = sid * NC + cid
    base = wid * EPW
    pltpu.sync_copy(zeros_hbm, stage)

    def zbody(t, carry):
        pltpu.sync_copy(stage, acc.at[pl.ds(sid * RPT + t * SR, SR)])
        return carry

    lax.fori_loop(0, RPT // SR, zbody, 0)
    pltpu.sync_copy(gidx_hbm.at[pl.ds(base, EPW)], idxall)
    pltpu.sync_copy(col_hbm.at[wid], colall)
    plsc.subcore_barrier()

    def gather(j, rows, sem):
        return pltpu.async_copy(z_hbm.at[idxall.at[pl.ds(j * CH, CH)]], rows, sem)

    def scatter(j, rows):
        pltpu.sync_copy(rows, acc.at[colall.at[j]], add=True)

    gather(0, rows0, sem0)

    def body(g, carry):
        j = 2 * g
        pltpu.async_copy(z_hbm.at[idxall.at[pl.ds((j + 1) * CH, CH)]], rows1, sem1)
        pltpu.make_async_copy(z_hbm.at[idxall.at[pl.ds(j * CH, CH)]], rows0, sem0).wait()
        scatter(j, rows0)
        pltpu.async_copy(z_hbm.at[idxall.at[pl.ds((j + 2) * CH, CH)]], rows0, sem0)
        pltpu.make_async_copy(z_hbm.at[idxall.at[pl.ds((j + 1) * CH, CH)]], rows1, sem1).wait()
        scatter(j + 1, rows1)
        return carry

    lax.fori_loop(0, (NCHUNK - 1) // 2, body, 0)
    pltpu.make_async_copy(z_hbm.at[idxall.at[pl.ds((NCHUNK - 1) * CH, CH)]], rows0, sem0).wait()
    scatter(NCHUNK - 1, rows0)
    plsc.subcore_barrier()

    def wbody(t, carry):
        pltpu.sync_copy(acc.at[pl.ds(sid * RPT + t * SR, SR)], stage)
        pltpu.sync_copy(stage, out_hbm.at[pl.ds(cid * NP + sid * RPT + t * SR, SR)])
        return carry

    lax.fori_loop(0, RPT // SR, wbody, 0)


_BR = 400


def _t1_body(h_ref, w_ref, b_ref, root_ref, dis_ref, inv_ref, z_ref, self_ref):
    x = lax.dot_general(h_ref[...], w_ref[...], (((1,), (1,)), ((), ())),
                        preferred_element_type=jnp.float32) + b_ref[...]
    self_ref[...] = jnp.maximum(x + root_ref[...], 0.0) * inv_ref[...]
    aval = lax.broadcasted_iota(jnp.int32, (_BR, A, D), 1).astype(jnp.float32)
    dis = dis_ref[...]
    z_ref[...] = jnp.maximum(x[:, None, :] + aval, 0.0) * dis[:, :, None]


def _t1(h, w, b, root, dis, inv):
    grid = (N // _BR,)
    return pl.pallas_call(
        _t1_body,
        grid=grid,
        in_specs=[
            pl.BlockSpec((_BR, D), lambda i: (i, 0)),
            pl.BlockSpec((D, D), lambda i: (0, 0)),
            pl.BlockSpec((1, D), lambda i: (0, 0)),
            pl.BlockSpec((1, D), lambda i: (0, 0)),
            pl.BlockSpec((_BR, 1), lambda i: (i, 0)),
            pl.BlockSpec((_BR, 1), lambda i: (i, 0)),
        ],
        out_specs=[
            pl.BlockSpec((_BR, A, D), lambda i: (i, 0, 0)),
            pl.BlockSpec((_BR, D), lambda i: (i, 0)),
        ],
        out_shape=[
            jax.ShapeDtypeStruct((N, A, D), jnp.float32),
            jax.ShapeDtypeStruct((N, D), jnp.float32),
        ],
    )(h, w, b, root, dis, inv)


def _t2_body(p0_ref, p1_ref, dis_ref, self_ref, g_ref, be_ref, o_ref, *, relu):
    y = (p0_ref[...] + p1_ref[...]) * dis_ref[...] + self_ref[...]
    y = y * (g_ref[...] * _BN_SCALE) + be_ref[...]
    if relu:
        y = jnp.maximum(y, 0.0)
    o_ref[...] = y


_BR2 = 80


def _t2(p, dis, selft, g, be, relu):
    grid = (N // _BR2,)
    nb = NP // _BR2
    return pl.pallas_call(
        functools.partial(_t2_body, relu=relu),
        grid=grid,
        in_specs=[
            pl.BlockSpec((_BR2, D), lambda i: (i, 0)),
            pl.BlockSpec((_BR2, D), lambda i, _nb=nb: (i + _nb, 0)),
            pl.BlockSpec((_BR2, 1), lambda i: (i, 0)),
            pl.BlockSpec((_BR2, D), lambda i: (i, 0)),
            pl.BlockSpec((1, D), lambda i: (0, 0)),
            pl.BlockSpec((1, D), lambda i: (0, 0)),
        ],
        out_specs=pl.BlockSpec((_BR2, D), lambda i: (i, 0)),
        out_shape=jax.ShapeDtypeStruct((N, D), jnp.float32),
    )(p, p, dis, selft, g, be)


def kernel(x, edge_index, edge_attr, W0, b0, root0, g0, be0,
           W1, b1, root1, g1, be1):
    row = edge_index[0]
    col = edge_index[1]
    gidx = row * A + edge_attr[:, 0]

    zerosD = jnp.zeros((SR, D), jnp.float32)
    col3 = col.reshape(NW, NCHUNK, CH)

    hist = _sc_hist(row)
    deg = jnp.sum(hist, axis=0)[:N] + 1.0
    dis = (deg ** -0.5).reshape(N, 1)
    inv = (1.0 / deg).reshape(N, 1)

    b0r = b0.reshape(1, D)
    g0r = g0.reshape(1, D)
    be0r = be0.reshape(1, D)
    b1r = b1.reshape(1, D)
    g1r = g1.reshape(1, D)
    be1r = be1.reshape(1, D)

    z0, self0 = _t1(x, W0, b0r, root0, dis, inv)
    p0 = _sc_agg(z0.reshape(N * A, D), gidx, col3, zerosD)
    h1 = _t2(p0, dis, self0, g0r, be0r, relu=True)
    z1, self1 = _t1(h1, W1, b1r, root1, dis, inv)
    p1 = _sc_agg(z1.reshape(N * A, D), gidx, col3, zerosD)
    h2 = _t2(p1, dis, self1, g1r, be1r, relu=False)
    return h2


# speedup vs baseline: 17.4988x; 1.0454x over previous
import functools

import jax
import jax.numpy as jnp
from jax import lax
from jax.experimental import pallas as pl
from jax.experimental.pallas import tpu as pltpu
from jax.experimental.pallas import tpu_sc as plsc

N = 10000
E = 320000
D = 128
A = 8

NC = 2
NS = 16
NW = NC * NS
EPW = E // NW
CH = 80
NCHUNK = EPW // CH
NP = 10240
RPT = NP // NS
SR = 32

_BN_SCALE = (1.0 + 1e-5) ** -0.5

_sc_mesh = plsc.VectorSubcoreMesh(core_axis_name="c", subcore_axis_name="s")


@functools.partial(
    pl.kernel,
    out_type=jax.ShapeDtypeStruct((NW, NP), jnp.float32),
    mesh=_sc_mesh,
    scratch_types=[
        pltpu.VMEM((NP,), jnp.float32),
        pltpu.VMEM((EPW,), jnp.int32),
    ],
    compiler_params=pltpu.CompilerParams(needs_layout_passes=False),
)
def _sc_hist(row_hbm, out_hbm, hist, idxs):
    cid = lax.axis_index("c")
    sid = lax.axis_index("s")
    wid = sid * NC + cid
    zero16 = jnp.zeros((16,), jnp.float32)

    def zbody(k, carry):
        hist[pl.ds(k * 16, 16)] = zero16
        return carry

    lax.fori_loop(0, NP // 16, zbody, 0)
    pltpu.sync_copy(row_hbm.at[pl.ds(wid * EPW, EPW)], idxs)
    ones16 = jnp.ones((16,), jnp.float32)

    def body(k, carry):
        idxv = idxs[pl.ds(k * 16, 16)]
        plsc.addupdate_scatter(hist, [idxv], ones16)
        return carry

    lax.fori_loop(0, EPW // 16, body, 0)
    pltpu.sync_copy(hist, out_hbm.at[wid])


@functools.partial(
    pl.kernel,
    out_type=jax.ShapeDtypeStruct((2 * NP, D), jnp.float32),
    mesh=_sc_mesh,
    scratch_types=[
        pltpu.VMEM_SHARED((NP, D), jnp.float32),
        pltpu.VMEM((SR, D), jnp.float32),
        pltpu.VMEM((EPW,), jnp.int32),
        pltpu.VMEM((NCHUNK, CH), jnp.int32),
        pltpu.VMEM((CH, D), jnp.float32),
        pltpu.VMEM((CH, D), jnp.float32),
        pltpu.SemaphoreType.DMA,
        pltpu.SemaphoreType.DMA,
    ],
    compiler_params=pltpu.CompilerParams(use_tc_tiling_on_sc=False),
)
def _sc_agg(z_hbm, gidx_hbm, col_hbm, zeros_hbm, out_hbm,
            acc, stage, idxall, colall, rows0, rows1, sem0, sem1):
    cid = lax.axis_index("c")
    sid = lax.axis_index("s")
    wid = sid * NC + cid
    base = wid * EPW
    pltpu.sync_copy(zeros_hbm, stage)

    def zbody(t, carry):
        pltpu.sync_copy(stage, acc.at[pl.ds(sid * RPT + t * SR, SR)])
        return carry

    lax.fori_loop(0, RPT // SR, zbody, 0)
    pltpu.sync_copy(gidx_hbm.at[pl.ds(base, EPW)], idxall)
    pltpu.sync_copy(col_hbm.at[wid], colall)
    plsc.subcore_barrier()

    def gather(j, rows, sem):
        return pltpu.async_copy(z_hbm.at[idxall.at[pl.ds(j * CH, CH)]], rows, sem)

    def scatter(j, rows):
        pltpu.sync_copy(rows, acc.at[colall.at[j]], add=True)

    gather(0, rows0, sem0)

    def body(g, carry):
        j = 2 * g
        pltpu.async_copy(z_hbm.at[idxall.at[pl.ds((j + 1) * CH, CH)]], rows1, sem1)
        pltpu.make_async_copy(z_hbm.at[idxall.at[pl.ds(j * CH, CH)]], rows0, sem0).wait()
        scatter(j, rows0)
        pltpu.async_copy(z_hbm.at[idxall.at[pl.ds((j + 2) * CH, CH)]], rows0, sem0)
        pltpu.make_async_copy(z_hbm.at[idxall.at[pl.ds((j + 1) * CH, CH)]], rows1, sem1).wait()
        scatter(j + 1, rows1)
        return carry

    lax.fori_loop(0, (NCHUNK - 1) // 2, body, 0)
    pltpu.make_async_copy(z_hbm.at[idxall.at[pl.ds((NCHUNK - 1) * CH, CH)]], rows0, sem0).wait()
    scatter(NCHUNK - 1, rows0)
    plsc.subcore_barrier()

    def wbody(t, carry):
        pltpu.sync_copy(acc.at[pl.ds(sid * RPT + t * SR, SR)], stage)
        pltpu.sync_copy(stage, out_hbm.at[pl.ds(cid * NP + sid * RPT + t * SR, SR)])
        return carry

    lax.fori_loop(0, RPT // SR, wbody, 0)


_BR = 400
_BR2 = 80


def _deg_dis_inv(hist_blk):
    deg = jnp.sum(hist_blk, axis=1) + 1.0
    return deg, lax.rsqrt(deg), 1.0 / deg


def _table_and_self(x, root, dis, inv, br, z_ref, self_ref):
    self_ref[...] = jnp.maximum(x + root, 0.0) * inv[:, None]
    aval = lax.broadcasted_iota(jnp.int32, (br, A, D), 1).astype(jnp.float32)
    z = jnp.maximum(x[:, None, :] + aval, 0.0) * dis[:, None, None]
    z_ref[...] = z.reshape(br * A, D)


def _k_pre_body(h_ref, w_ref, b_ref, root_ref, hist_ref, z_ref, self_ref):
    _, dis, inv = _deg_dis_inv(hist_ref[...])
    x = lax.dot_general(h_ref[...], w_ref[...], (((1,), (1,)), ((), ())),
                        preferred_element_type=jnp.float32) + b_ref[...]
    _table_and_self(x, root_ref[...], dis, inv, _BR, z_ref, self_ref)


def _k_pre(h, w, b, root, hist):
    return pl.pallas_call(
        _k_pre_body,
        grid=(N // _BR,),
        in_specs=[
            pl.BlockSpec((_BR, D), lambda i: (i, 0)),
            pl.BlockSpec((D, D), lambda i: (0, 0)),
            pl.BlockSpec((1, D), lambda i: (0, 0)),
            pl.BlockSpec((1, D), lambda i: (0, 0)),
            pl.BlockSpec((_BR, NW), lambda i: (i, 0)),
        ],
        out_specs=[
            pl.BlockSpec((_BR * A, D), lambda i: (i, 0)),
            pl.BlockSpec((_BR, D), lambda i: (i, 0)),
        ],
        out_shape=[
            jax.ShapeDtypeStruct((N * A, D), jnp.float32),
            jax.ShapeDtypeStruct((N, D), jnp.float32),
        ],
    )(h, w, b, root, hist)


def _combine(p0, p1, dis, selft, g, be):
    y = (p0 + p1) * dis[:, None] + selft
    return y * (g * _BN_SCALE) + be


def _k_mid_body(p0_ref, p1_ref, self0_ref, g_ref, be_ref, hist_ref,
                w_ref, b_ref, root_ref, z_ref, self_ref):
    _, dis, inv = _deg_dis_inv(hist_ref[...])
    h1 = jnp.maximum(
        _combine(p0_ref[...], p1_ref[...], dis, self0_ref[...],
                 g_ref[...], be_ref[...]), 0.0)
    x = lax.dot_general(h1, w_ref[...], (((1,), (1,)), ((), ())),
                        preferred_element_type=jnp.float32) + b_ref[...]
    _table_and_self(x, root_ref[...], dis, inv, _BR2, z_ref, self_ref)


def _k_mid(p, self0, g, be, hist, w, b, root):
    nb = NP // _BR2
    return pl.pallas_call(
        _k_mid_body,
        grid=(N // _BR2,),
        in_specs=[
            pl.BlockSpec((_BR2, D), lambda i: (i, 0)),
            pl.BlockSpec((_BR2, D), lambda i, _nb=nb: (i + _nb, 0)),
            pl.BlockSpec((_BR2, D), lambda i: (i, 0)),
            pl.BlockSpec((1, D), lambda i: (0, 0)),
            pl.BlockSpec((1, D), lambda i: (0, 0)),
            pl.BlockSpec((_BR2, NW), lambda i: (i, 0)),
            pl.BlockSpec((D, D), lambda i: (0, 0)),
            pl.BlockSpec((1, D), lambda i: (0, 0)),
            pl.BlockSpec((1, D), lambda i: (0, 0)),
        ],
        out_specs=[
            pl.BlockSpec((_BR2 * A, D), lambda i: (i, 0)),
            pl.BlockSpec((_BR2, D), lambda i: (i, 0)),
        ],
        out_shape=[
            jax.ShapeDtypeStruct((N * A, D), jnp.float32),
            jax.ShapeDtypeStruct((N, D), jnp.float32),
        ],
    )(p, p, self0, g, be, hist, w, b, root)


def _k_post_body(p0_ref, p1_ref, self1_ref, g_ref, be_ref, hist_ref, o_ref):
    _, dis, _ = _deg_dis_inv(hist_ref[...])
    o_ref[...] = _combine(p0_ref[...], p1_ref[...], dis, self1_ref[...],
                          g_ref[...], be_ref[...])


def _k_post(p, self1, g, be, hist):
    nb = NP // _BR2
    return pl.pallas_call(
        _k_post_body,
        grid=(N // _BR2,),
        in_specs=[
            pl.BlockSpec((_BR2, D), lambda i: (i, 0)),
            pl.BlockSpec((_BR2, D), lambda i, _nb=nb: (i + _nb, 0)),
            pl.BlockSpec((_BR2, D), lambda i: (i, 0)),
            pl.BlockSpec((1, D), lambda i: (0, 0)),
            pl.BlockSpec((1, D), lambda i: (0, 0)),
            pl.BlockSpec((_BR2, NW), lambda i: (i, 0)),
        ],
        out_specs=pl.BlockSpec((_BR2, D), lambda i: (i, 0)),
        out_shape=jax.ShapeDtypeStruct((N, D), jnp.float32),
    )(p, p, self1, g, be, hist)


def kernel(x, edge_index, edge_attr, W0, b0, root0, g0, be0,
           W1, b1, root1, g1, be1):
    row = edge_index[0]
    col = edge_index[1]
    gidx = row * A + edge_attr[:, 0]

    zerosD = jnp.zeros((SR, D), jnp.float32)
    col3 = col.reshape(NW, NCHUNK, CH)

    hist = _sc_hist(row).T

    b0r = b0.reshape(1, D)
    b1r = b1.reshape(1, D)

    z0, self0 = _k_pre(x, W0, b0r, root0, hist)
    p0 = _sc_agg(z0, gidx, col3, zerosD)
    z1, self1 = _k_mid(p0, self0, g0.reshape(1, D), be0.reshape(1, D),
                       hist, W1, b1r, root1)
    p1 = _sc_agg(z1, gidx, col3, zerosD)
    return _k_post(p1, self1, g1.reshape(1, D), be1.reshape(1, D), hist)


# speedup vs baseline: 23.8905x; 1.3653x over previous
import functools

import jax
import jax.numpy as jnp
from jax import lax
from jax.experimental import pallas as pl
from jax.experimental.pallas import tpu as pltpu
from jax.experimental.pallas import tpu_sc as plsc

N = 10000
E = 320000
D = 128
A = 8

NC = 2
NS = 16
NW = NC * NS
EPW = E // NW
CH = 80
NCHUNK = EPW // CH
NP = 10240
RPT = NP // NS
SR = 32

_BN_SCALE = (1.0 + 1e-5) ** -0.5

_sc_mesh = plsc.VectorSubcoreMesh(core_axis_name="c", subcore_axis_name="s")


@functools.partial(
    pl.kernel,
    out_type=jax.ShapeDtypeStruct((NW, NP), jnp.float32),
    mesh=_sc_mesh,
    scratch_types=[
        pltpu.VMEM((NP,), jnp.float32),
        pltpu.VMEM((EPW,), jnp.int32),
    ],
    compiler_params=pltpu.CompilerParams(needs_layout_passes=False),
)
def _sc_hist(row_hbm, out_hbm, hist, idxs):
    cid = lax.axis_index("c")
    sid = lax.axis_index("s")
    wid = sid * NC + cid
    zero16 = jnp.zeros((16,), jnp.float32)

    def zbody(k, carry):
        hist[pl.ds(k * 16, 16)] = zero16
        return carry

    lax.fori_loop(0, NP // 16, zbody, 0)
    pltpu.sync_copy(row_hbm.at[pl.ds(wid * EPW, EPW)], idxs)
    ones16 = jnp.ones((16,), jnp.float32)

    def body(k, carry):
        idxv = idxs[pl.ds(k * 16, 16)]
        plsc.addupdate_scatter(hist, [idxv], ones16)
        return carry

    lax.fori_loop(0, EPW // 16, body, 0)
    pltpu.sync_copy(hist, out_hbm.at[wid])


@functools.partial(
    pl.kernel,
    out_type=[jax.ShapeDtypeStruct((NP, D), jnp.float32),
              jax.ShapeDtypeStruct((NP, D), jnp.float32)],
    mesh=_sc_mesh,
    scratch_types=[
        pltpu.VMEM_SHARED((NP, D), jnp.float32),
        pltpu.VMEM((SR, D), jnp.float32),
        pltpu.VMEM((EPW,), jnp.int32),
        pltpu.VMEM((NCHUNK, CH), jnp.int32),
        pltpu.VMEM((CH, D), jnp.float32),
        pltpu.VMEM((CH, D), jnp.float32),
        pltpu.SemaphoreType.DMA,
        pltpu.SemaphoreType.DMA,
    ],
    compiler_params=pltpu.CompilerParams(use_tc_tiling_on_sc=False),
)
def _sc_agg(z_hbm, gidx_hbm, col_hbm, zeros_hbm, out0_hbm, out1_hbm,
            acc, stage, idxall, colall, rows0, rows1, sem0, sem1):
    cid = lax.axis_index("c")
    sid = lax.axis_index("s")
    wid = sid * NC + cid
    base = wid * EPW
    pltpu.sync_copy(zeros_hbm, stage)

    def zbody(t, carry):
        pltpu.sync_copy(stage, acc.at[pl.ds(sid * RPT + t * SR, SR)])
        return carry

    lax.fori_loop(0, RPT // SR, zbody, 0)
    pltpu.sync_copy(gidx_hbm.at[pl.ds(base, EPW)], idxall)
    pltpu.sync_copy(col_hbm.at[wid], colall)
    plsc.subcore_barrier()

    def gather(j, rows, sem):
        return pltpu.async_copy(z_hbm.at[idxall.at[pl.ds(j * CH, CH)]], rows, sem)

    def scatter(j, rows):
        pltpu.sync_copy(rows, acc.at[colall.at[j]], add=True)

    gather(0, rows0, sem0)

    def body(g, carry):
        j = 2 * g
        pltpu.async_copy(z_hbm.at[idxall.at[pl.ds((j + 1) * CH, CH)]], rows1, sem1)
        pltpu.make_async_copy(z_hbm.at[idxall.at[pl.ds(j * CH, CH)]], rows0, sem0).wait()
        scatter(j, rows0)
        pltpu.async_copy(z_hbm.at[idxall.at[pl.ds((j + 2) * CH, CH)]], rows0, sem0)
        pltpu.make_async_copy(z_hbm.at[idxall.at[pl.ds((j + 1) * CH, CH)]], rows1, sem1).wait()
        scatter(j + 1, rows1)
        return carry

    lax.fori_loop(0, (NCHUNK - 1) // 2, body, 0)
    pltpu.make_async_copy(z_hbm.at[idxall.at[pl.ds((NCHUNK - 1) * CH, CH)]], rows0, sem0).wait()
    scatter(NCHUNK - 1, rows0)
    plsc.subcore_barrier()

    def wbody(t, carry):
        pltpu.sync_copy(acc.at[pl.ds(sid * RPT + t * SR, SR)], stage)

        @pl.when(cid == 0)
        def _():
            pltpu.sync_copy(stage, out0_hbm.at[pl.ds(sid * RPT + t * SR, SR)])

        @pl.when(cid == 1)
        def _():
            pltpu.sync_copy(stage, out1_hbm.at[pl.ds(sid * RPT + t * SR, SR)])

        return carry

    lax.fori_loop(0, RPT // SR, wbody, 0)


_BR = 1000
_BR2 = 1000


def _deg_dis_inv(hist_blk):
    deg = jnp.sum(hist_blk, axis=1) + 1.0
    return deg, lax.rsqrt(deg), 1.0 / deg


def _table_and_self(x, root, dis, inv, br, z_ref, self_ref):
    self_ref[...] = jnp.maximum(x + root, 0.0) * inv[:, None]
    aval = lax.broadcasted_iota(jnp.int32, (br, A, D), 1).astype(jnp.float32)
    z = jnp.maximum(x[:, None, :] + aval, 0.0) * dis[:, None, None]
    z_ref[...] = z.reshape(br * A, D)


def _k_pre_body(h_ref, w_ref, b_ref, root_ref, hist_ref, z_ref, self_ref):
    _, dis, inv = _deg_dis_inv(hist_ref[...])
    x = lax.dot_general(h_ref[...], w_ref[...], (((1,), (1,)), ((), ())),
                        preferred_element_type=jnp.float32) + b_ref[...]
    _table_and_self(x, root_ref[...], dis, inv, _BR, z_ref, self_ref)


def _k_pre(h, w, b, root, hist):
    return pl.pallas_call(
        _k_pre_body,
        grid=(N // _BR,),
        in_specs=[
            pl.BlockSpec((_BR, D), lambda i: (i, 0)),
            pl.BlockSpec((D, D), lambda i: (0, 0)),
            pl.BlockSpec((1, D), lambda i: (0, 0)),
            pl.BlockSpec((1, D), lambda i: (0, 0)),
            pl.BlockSpec((_BR, NW), lambda i: (i, 0)),
        ],
        out_specs=[
            pl.BlockSpec((_BR * A, D), lambda i: (i, 0)),
            pl.BlockSpec((_BR, D), lambda i: (i, 0)),
        ],
        out_shape=[
            jax.ShapeDtypeStruct((N * A, D), jnp.float32),
            jax.ShapeDtypeStruct((N, D), jnp.float32),
        ],
    )(h, w, b, root, hist)


def _combine(p0, p1, dis, selft, g, be):
    y = (p0 + p1) * dis[:, None] + selft
    return y * (g * _BN_SCALE) + be


def _k_mid_body(p0_ref, p1_ref, self0_ref, g_ref, be_ref, hist_ref,
                w_ref, b_ref, root_ref, z_ref, self_ref):
    _, dis, inv = _deg_dis_inv(hist_ref[...])
    h1 = jnp.maximum(
        _combine(p0_ref[...], p1_ref[...], dis, self0_ref[...],
                 g_ref[...], be_ref[...]), 0.0)
    x = lax.dot_general(h1, w_ref[...], (((1,), (1,)), ((), ())),
                        preferred_element_type=jnp.float32) + b_ref[...]
    _table_and_self(x, root_ref[...], dis, inv, _BR2, z_ref, self_ref)


def _k_mid(pa, pb, self0, g, be, hist, w, b, root):
    return pl.pallas_call(
        _k_mid_body,
        grid=(N // _BR2,),
        in_specs=[
            pl.BlockSpec((_BR2, D), lambda i: (i, 0)),
            pl.BlockSpec((_BR2, D), lambda i: (i, 0)),
            pl.BlockSpec((_BR2, D), lambda i: (i, 0)),
            pl.BlockSpec((1, D), lambda i: (0, 0)),
            pl.BlockSpec((1, D), lambda i: (0, 0)),
            pl.BlockSpec((_BR2, NW), lambda i: (i, 0)),
            pl.BlockSpec((D, D), lambda i: (0, 0)),
            pl.BlockSpec((1, D), lambda i: (0, 0)),
            pl.BlockSpec((1, D), lambda i: (0, 0)),
        ],
        out_specs=[
            pl.BlockSpec((_BR2 * A, D), lambda i: (i, 0)),
            pl.BlockSpec((_BR2, D), lambda i: (i, 0)),
        ],
        out_shape=[
            jax.ShapeDtypeStruct((N * A, D), jnp.float32),
            jax.ShapeDtypeStruct((N, D), jnp.float32),
        ],
    )(pa, pb, self0, g, be, hist, w, b, root)


def _k_post_body(p0_ref, p1_ref, self1_ref, g_ref, be_ref, hist_ref, o_ref):
    _, dis, _ = _deg_dis_inv(hist_ref[...])
    o_ref[...] = _combine(p0_ref[...], p1_ref[...], dis, self1_ref[...],
                          g_ref[...], be_ref[...])


def _k_post(pa, pb, self1, g, be, hist):
    return pl.pallas_call(
        _k_post_body,
        grid=(N // _BR2,),
        in_specs=[
            pl.BlockSpec((_BR2, D), lambda i: (i, 0)),
            pl.BlockSpec((_BR2, D), lambda i: (i, 0)),
            pl.BlockSpec((_BR2, D), lambda i: (i, 0)),
            pl.BlockSpec((1, D), lambda i: (0, 0)),
            pl.BlockSpec((1, D), lambda i: (0, 0)),
            pl.BlockSpec((_BR2, NW), lambda i: (i, 0)),
        ],
        out_specs=pl.BlockSpec((_BR2, D), lambda i: (i, 0)),
        out_shape=jax.ShapeDtypeStruct((N, D), jnp.float32),
    )(pa, pb, self1, g, be, hist)


def kernel(x, edge_index, edge_attr, W0, b0, root0, g0, be0,
           W1, b1, root1, g1, be1):
    row = edge_index[0]
    col = edge_index[1]
    gidx = row * A + edge_attr[:, 0]

    zerosD = jnp.zeros((SR, D), jnp.float32)
    col3 = col.reshape(NW, NCHUNK, CH)

    hist = _sc_hist(row).T

    b0r = b0.reshape(1, D)
    b1r = b1.reshape(1, D)

    z0, self0 = _k_pre(x, W0, b0r, root0, hist)
    p0a, p0b = _sc_agg(z0, gidx, col3, zerosD)
    z1, self1 = _k_mid(p0a, p0b, self0, g0.reshape(1, D), be0.reshape(1, D),
                       hist, W1, b1r, root1)
    p1a, p1b = _sc_agg(z1, gidx, col3, zerosD)
    return _k_post(p1a, p1b, self1, g1.reshape(1, D), be1.reshape(1, D), hist)


# speedup vs baseline: 24.0989x; 1.0087x over previous
import functools

import jax
import jax.numpy as jnp
from jax import lax
from jax.experimental import pallas as pl
from jax.experimental.pallas import tpu as pltpu
from jax.experimental.pallas import tpu_sc as plsc

N = 10000
E = 320000
D = 128
A = 8

NC = 2
NS = 16
NW = NC * NS
EPW = E // NW
CH = 40
NCHUNK = EPW // CH
NP = 10240
RPT = NP // NS
SR = 32

_BN_SCALE = (1.0 + 1e-5) ** -0.5

_sc_mesh = plsc.VectorSubcoreMesh(core_axis_name="c", subcore_axis_name="s")


@functools.partial(
    pl.kernel,
    out_type=jax.ShapeDtypeStruct((NW, NP), jnp.float32),
    mesh=_sc_mesh,
    scratch_types=[
        pltpu.VMEM((NP,), jnp.float32),
        pltpu.VMEM((EPW,), jnp.int32),
    ],
    compiler_params=pltpu.CompilerParams(needs_layout_passes=False),
)
def _sc_hist(row_hbm, out_hbm, hist, idxs):
    cid = lax.axis_index("c")
    sid = lax.axis_index("s")
    wid = sid * NC + cid
    zero16 = jnp.zeros((16,), jnp.float32)

    def zbody(k, carry):
        hist[pl.ds(k * 16, 16)] = zero16
        return carry

    lax.fori_loop(0, NP // 16, zbody, 0)
    pltpu.sync_copy(row_hbm.at[pl.ds(wid * EPW, EPW)], idxs)
    ones16 = jnp.ones((16,), jnp.float32)

    def body(k, carry):
        idxv = idxs[pl.ds(k * 16, 16)]
        plsc.addupdate_scatter(hist, [idxv], ones16)
        return carry

    lax.fori_loop(0, EPW // 16, body, 0)
    pltpu.sync_copy(hist, out_hbm.at[wid])


@functools.partial(
    pl.kernel,
    out_type=[jax.ShapeDtypeStruct((NP, D), jnp.float32),
              jax.ShapeDtypeStruct((NP, D), jnp.float32)],
    mesh=_sc_mesh,
    scratch_types=[
        pltpu.VMEM_SHARED((NP, D), jnp.float32),
        pltpu.VMEM((SR, D), jnp.float32),
        pltpu.VMEM((EPW,), jnp.int32),
        pltpu.VMEM((NCHUNK, CH), jnp.int32),
        pltpu.VMEM((CH, D), jnp.float32),
        pltpu.VMEM((CH, D), jnp.float32),
        pltpu.VMEM((CH, D), jnp.float32),
        pltpu.SemaphoreType.DMA,
        pltpu.SemaphoreType.DMA,
        pltpu.SemaphoreType.DMA,
    ],
    compiler_params=pltpu.CompilerParams(use_tc_tiling_on_sc=False),
)
def _sc_agg(z_hbm, gidx_hbm, col_hbm, zeros_hbm, out0_hbm, out1_hbm,
            acc, stage, idxall, colall, rows0, rows1, rows2, sem0, sem1, sem2):
    cid = lax.axis_index("c")
    sid = lax.axis_index("s")
    wid = sid * NC + cid
    base = wid * EPW
    pltpu.sync_copy(zeros_hbm, stage)

    def zbody(t, carry):
        pltpu.sync_copy(stage, acc.at[pl.ds(sid * RPT + t * SR, SR)])
        return carry

    lax.fori_loop(0, RPT // SR, zbody, 0)
    pltpu.sync_copy(gidx_hbm.at[pl.ds(base, EPW)], idxall)
    pltpu.sync_copy(col_hbm.at[wid], colall)
    plsc.subcore_barrier()

    def gather(j, rows, sem):
        return pltpu.async_copy(z_hbm.at[idxall.at[pl.ds(j * CH, CH)]], rows, sem)

    def gwait(j, rows, sem):
        pltpu.make_async_copy(z_hbm.at[idxall.at[pl.ds(j * CH, CH)]], rows, sem).wait()

    def scatter(j, rows):
        pltpu.sync_copy(rows, acc.at[colall.at[j]], add=True)

    gather(0, rows0, sem0)
    gather(1, rows1, sem1)
    gather(2, rows2, sem2)
    bufs = ((rows0, sem0), (rows1, sem1), (rows2, sem2))

    def body(g, carry):
        j = 3 * g
        for b, (rows, sem) in enumerate(bufs):
            gwait(j + b, rows, sem)
            scatter(j + b, rows)
            nxt = j + b + 3

            @pl.when(nxt < NCHUNK)
            def _():
                gather(nxt, rows, sem)

        return carry

    lax.fori_loop(0, NCHUNK // 3, body, 0)
    gwait(NCHUNK - 1, rows0, sem0)
    scatter(NCHUNK - 1, rows0)
    plsc.subcore_barrier()

    def wbody(t, carry):
        pltpu.sync_copy(acc.at[pl.ds(sid * RPT + t * SR, SR)], stage)

        @pl.when(cid == 0)
        def _():
            pltpu.sync_copy(stage, out0_hbm.at[pl.ds(sid * RPT + t * SR, SR)])

        @pl.when(cid == 1)
        def _():
            pltpu.sync_copy(stage, out1_hbm.at[pl.ds(sid * RPT + t * SR, SR)])

        return carry

    lax.fori_loop(0, RPT // SR, wbody, 0)


_BR = 1000
_BR2 = 1000


def _deg_dis_inv(hist_blk):
    deg = jnp.sum(hist_blk, axis=1) + 1.0
    return deg, lax.rsqrt(deg), 1.0 / deg


def _table_and_self(x, root, dis, inv, br, z_ref, self_ref):
    self_ref[...] = jnp.maximum(x + root, 0.0) * inv[:, None]
    aval = lax.broadcasted_iota(jnp.int32, (br, A, D), 1).astype(jnp.float32)
    z = jnp.maximum(x[:, None, :] + aval, 0.0) * dis[:, None, None]
    z_ref[...] = z.reshape(br * A, D)


def _k_pre_body(h_ref, w_ref, b_ref, root_ref, hist_ref, z_ref, self_ref):
    _, dis, inv = _deg_dis_inv(hist_ref[...])
    x = lax.dot_general(h_ref[...], w_ref[...], (((1,), (1,)), ((), ())),
                        preferred_element_type=jnp.float32) + b_ref[...]
    _table_and_self(x, root_ref[...], dis, inv, _BR, z_ref, self_ref)


def _k_pre(h, w, b, root, hist):
    return pl.pallas_call(
        _k_pre_body,
        grid=(N // _BR,),
        in_specs=[
            pl.BlockSpec((_BR, D), lambda i: (i, 0)),
            pl.BlockSpec((D, D), lambda i: (0, 0)),
            pl.BlockSpec((1, D), lambda i: (0, 0)),
            pl.BlockSpec((1, D), lambda i: (0, 0)),
            pl.BlockSpec((_BR, NW), lambda i: (i, 0)),
        ],
        out_specs=[
            pl.BlockSpec((_BR * A, D), lambda i: (i, 0)),
            pl.BlockSpec((_BR, D), lambda i: (i, 0)),
        ],
        out_shape=[
            jax.ShapeDtypeStruct((N * A, D), jnp.float32),
            jax.ShapeDtypeStruct((N, D), jnp.float32),
        ],
    )(h, w, b, root, hist)


def _combine(p0, p1, dis, selft, g, be):
    y = (p0 + p1) * dis[:, None] + selft
    return y * (g * _BN_SCALE) + be


def _k_mid_body(p0_ref, p1_ref, self0_ref, g_ref, be_ref, hist_ref,
                w_ref, b_ref, root_ref, z_ref, self_ref):
    _, dis, inv = _deg_dis_inv(hist_ref[...])
    h1 = jnp.maximum(
        _combine(p0_ref[...], p1_ref[...], dis, self0_ref[...],
                 g_ref[...], be_ref[...]), 0.0)
    x = lax.dot_general(h1, w_ref[...], (((1,), (1,)), ((), ())),
                        preferred_element_type=jnp.float32) + b_ref[...]
    _table_and_self(x, root_ref[...], dis, inv, _BR2, z_ref, self_ref)


def _k_mid(pa, pb, self0, g, be, hist, w, b, root):
    return pl.pallas_call(
        _k_mid_body,
        grid=(N // _BR2,),
        in_specs=[
            pl.BlockSpec((_BR2, D), lambda i: (i, 0)),
            pl.BlockSpec((_BR2, D), lambda i: (i, 0)),
            pl.BlockSpec((_BR2, D), lambda i: (i, 0)),
            pl.BlockSpec((1, D), lambda i: (0, 0)),
            pl.BlockSpec((1, D), lambda i: (0, 0)),
            pl.BlockSpec((_BR2, NW), lambda i: (i, 0)),
            pl.BlockSpec((D, D), lambda i: (0, 0)),
            pl.BlockSpec((1, D), lambda i: (0, 0)),
            pl.BlockSpec((1, D), lambda i: (0, 0)),
        ],
        out_specs=[
            pl.BlockSpec((_BR2 * A, D), lambda i: (i, 0)),
            pl.BlockSpec((_BR2, D), lambda i: (i, 0)),
        ],
        out_shape=[
            jax.ShapeDtypeStruct((N * A, D), jnp.float32),
            jax.ShapeDtypeStruct((N, D), jnp.float32),
        ],
    )(pa, pb, self0, g, be, hist, w, b, root)


def _k_post_body(p0_ref, p1_ref, self1_ref, g_ref, be_ref, hist_ref, o_ref):
    _, dis, _ = _deg_dis_inv(hist_ref[...])
    o_ref[...] = _combine(p0_ref[...], p1_ref[...], dis, self1_ref[...],
                          g_ref[...], be_ref[...])


def _k_post(pa, pb, self1, g, be, hist):
    return pl.pallas_call(
        _k_post_body,
        grid=(N // _BR2,),
        in_specs=[
            pl.BlockSpec((_BR2, D), lambda i: (i, 0)),
            pl.BlockSpec((_BR2, D), lambda i: (i, 0)),
            pl.BlockSpec((_BR2, D), lambda i: (i, 0)),
            pl.BlockSpec((1, D), lambda i: (0, 0)),
            pl.BlockSpec((1, D), lambda i: (0, 0)),
            pl.BlockSpec((_BR2, NW), lambda i: (i, 0)),
        ],
        out_specs=pl.BlockSpec((_BR2, D), lambda i: (i, 0)),
        out_shape=jax.ShapeDtypeStruct((N, D), jnp.float32),
    )(pa, pb, self1, g, be, hist)


def kernel(x, edge_index, edge_attr, W0, b0, root0, g0, be0,
           W1, b1, root1, g1, be1):
    row = edge_index[0]
    col = edge_index[1]
    gidx = row * A + edge_attr[:, 0]

    zerosD = jnp.zeros((SR, D), jnp.float32)
    col3 = col.reshape(NW, NCHUNK, CH)

    hist = _sc_hist(row).T

    b0r = b0.reshape(1, D)
    b1r = b1.reshape(1, D)

    z0, self0 = _k_pre(x, W0, b0r, root0, hist)
    p0a, p0b = _sc_agg(z0, gidx, col3, zerosD)
    z1, self1 = _k_mid(p0a, p0b, self0, g0.reshape(1, D), be0.reshape(1, D),
                       hist, W1, b1r, root1)
    p1a, p1b = _sc_agg(z1, gidx, col3, zerosD)
    return _k_post(p1a, p1b, self1, g1.reshape(1, D), be1.reshape(1, D), hist)
